# Initial kernel scaffold; baseline (speedup 1.0000x reference)
import jax
import jax.numpy as jnp
from jax.experimental import pallas as pl


def kernel(x, edge_index, W1a, b1a, W1b, b1b, W2a, b2a, W2b, b2b, Wl, bl):
    raise NotImplementedError("write your pallas kernel here")


# speedup vs baseline: 7.3583x; 7.3583x over previous
import functools
import math

import jax
import jax.numpy as jnp
from jax import lax
from jax.experimental import pallas as pl
from jax.experimental.pallas import tpu as pltpu
from jax.experimental.pallas import tpu_sc as plsc

F32 = jnp.float32

NC = 2
NS = 16
NW = NC * NS
CHUNK = 128
STRIPE = 640
CNTW = 8


def _sc_aggregate(n_pad, d, k_chunks, with_counts):
    mesh = plsc.VectorSubcoreMesh(core_axis_name="c", subcore_axis_name="s")
    out_type = [jax.ShapeDtypeStruct((NC, n_pad, d), F32)]
    scratch = [
        pltpu.VMEM((k_chunks, CHUNK), jnp.int32),
        pltpu.VMEM((k_chunks, CHUNK), jnp.int32),
        pltpu.VMEM((CHUNK, d), F32),
        pltpu.VMEM_SHARED((n_pad, d), F32),
        pltpu.SemaphoreType.DMA,
    ]
    if with_counts:
        out_type.append(jax.ShapeDtypeStruct((NC, n_pad, CNTW), F32))
        scratch += [
            pltpu.VMEM((CHUNK, CNTW), F32),
            pltpu.VMEM_SHARED((n_pad, CNTW), F32),
        ]

    @functools.partial(
        pl.kernel, mesh=mesh, out_type=out_type, scratch_types=scratch,
        compiler_params=pltpu.CompilerParams(use_tc_tiling_on_sc=False))
    def sc_kernel(table_hbm, src_hbm, dst_hbm, zeros_d_hbm, *rest):
        if with_counts:
            (zeros_c_hbm, ones_hbm, acc_out, cnt_out,
             src_v, dst_v, rows_v, acc_sh, sem, ones_v, cnt_sh) = rest
        else:
            acc_out, src_v, dst_v, rows_v, acc_sh, sem = rest
        c = lax.axis_index("c")
        s = lax.axis_index("s")
        wid = s * NC + c
        base = s * STRIPE
        pltpu.sync_copy(zeros_d_hbm.at[pl.ds(base, STRIPE)],
                        acc_sh.at[pl.ds(base, STRIPE)])
        if with_counts:
            pltpu.sync_copy(zeros_c_hbm.at[pl.ds(base, STRIPE)],
                            cnt_sh.at[pl.ds(base, STRIPE)])
            pltpu.sync_copy(ones_hbm, ones_v)
        pltpu.sync_copy(src_hbm.at[wid], src_v)
        pltpu.sync_copy(dst_hbm.at[wid], dst_v)
        plsc.subcore_barrier()

        def body(j, carry):
            pltpu.async_copy(table_hbm.at[src_v.at[j]], rows_v, sem).wait()
            pltpu.sync_copy(rows_v, acc_sh.at[dst_v.at[j]], add=True)
            if with_counts:
                pltpu.sync_copy(ones_v, cnt_sh.at[dst_v.at[j]], add=True)
            return carry

        lax.fori_loop(0, k_chunks, body, 0)
        plsc.subcore_barrier()
        pltpu.sync_copy(acc_sh.at[pl.ds(base, STRIPE)],
                        acc_out.at[c, pl.ds(base, STRIPE)])
        if with_counts:
            pltpu.sync_copy(cnt_sh.at[pl.ds(base, STRIPE)],
                            cnt_out.at[c, pl.ds(base, STRIPE)])

    return sc_kernel


def _tc_table(x_pad, w, b, n_pad, blk):
    d_in, d_out = w.shape

    def body(x_ref, w_ref, b_ref, o_ref):
        o_ref[...] = jnp.dot(x_ref[...], w_ref[...],
                             preferred_element_type=F32) + b_ref[...]

    return pl.pallas_call(
        body,
        grid=(n_pad // blk,),
        in_specs=[
            pl.BlockSpec((blk, d_in), lambda i: (i, 0)),
            pl.BlockSpec((d_in, d_out), lambda i: (0, 0)),
            pl.BlockSpec((1, d_out), lambda i: (0, 0)),
        ],
        out_specs=pl.BlockSpec((blk, d_out), lambda i: (i, 0)),
        out_shape=jax.ShapeDtypeStruct((n_pad, d_out), F32),
    )(x_pad, w, b.reshape(1, d_out))


def _tc_update(h, acc, cnt, w_x, w_a, b, w_next, b_next, n_pad, blk):
    d_in = h.shape[1]
    d_agg = acc.shape[2]
    d_out = w_x.shape[1]
    has_next = w_next is not None
    d_next = w_next.shape[1] if has_next else 0

    def body(h_ref, p_ref, c_ref, wx_ref, wa_ref, b_ref, *rest):
        if has_next:
            wn_ref, bn_ref, o_ref, on_ref = rest
        else:
            (o_ref,) = rest
        agg_sum = p_ref[0] + p_ref[1]
        cnt_col = c_ref[0, :, :1] + c_ref[1, :, :1]
        agg = agg_sum * (1.0 / jnp.maximum(cnt_col, 1.0))
        hn = jnp.dot(h_ref[...], wx_ref[...], preferred_element_type=F32)
        hn = hn + jnp.dot(agg, wa_ref[...], preferred_element_type=F32)
        hn = jnp.maximum(hn + b_ref[...], 0.0)
        o_ref[...] = hn
        if has_next:
            on_ref[...] = jnp.dot(hn, wn_ref[...],
                                  preferred_element_type=F32) + bn_ref[...]

    in_specs = [
        pl.BlockSpec((blk, d_in), lambda i: (i, 0)),
        pl.BlockSpec((NC, blk, d_agg), lambda i: (0, i, 0)),
        pl.BlockSpec((NC, blk, CNTW), lambda i: (0, i, 0)),
        pl.BlockSpec((d_in, d_out), lambda i: (0, 0)),
        pl.BlockSpec((d_agg, d_out), lambda i: (0, 0)),
        pl.BlockSpec((1, d_out), lambda i: (0, 0)),
    ]
    args = [h, acc, cnt, w_x, w_a, b.reshape(1, d_out)]
    out_specs = [pl.BlockSpec((blk, d_out), lambda i: (i, 0))]
    out_shape = [jax.ShapeDtypeStruct((n_pad, d_out), F32)]
    if has_next:
        in_specs += [
            pl.BlockSpec((d_out, d_next), lambda i: (0, 0)),
            pl.BlockSpec((1, d_next), lambda i: (0, 0)),
        ]
        args += [w_next, b_next.reshape(1, d_next)]
        out_specs.append(pl.BlockSpec((blk, d_next), lambda i: (i, 0)))
        out_shape.append(jax.ShapeDtypeStruct((n_pad, d_next), F32))

    res = pl.pallas_call(
        body,
        grid=(n_pad // blk,),
        in_specs=in_specs,
        out_specs=out_specs,
        out_shape=out_shape,
    )(*args)
    return res if has_next else (res[0], None)


def kernel(x, edge_index, W1a, b1a, W1b, b1b, W2a, b2a, W2b, b2b, Wl, bl):
    n, d = x.shape
    e = edge_index.shape[1]
    n_pad = NS * STRIPE
    blk = n_pad // NS

    k_chunks = math.ceil(e / (NW * CHUNK))
    e_pad = NW * k_chunks * CHUNK
    src_r = jnp.concatenate(
        [edge_index[0], jnp.zeros((e_pad - e,), jnp.int32)]).reshape(
            NW, k_chunks, CHUNK)
    dst_r = jnp.concatenate(
        [edge_index[1], jnp.full((e_pad - e,), n, jnp.int32)]).reshape(
            NW, k_chunks, CHUNK)

    x_pad = jnp.pad(x, ((0, n_pad - n), (0, 0)))
    d1 = W1a.shape[1]
    d2 = W2a.shape[1]
    zeros1 = jnp.zeros((n_pad, d1), F32)
    zeros2 = jnp.zeros((n_pad, d2), F32)
    zeros_c = jnp.zeros((n_pad, CNTW), F32)
    ones = jnp.ones((CHUNK, CNTW), F32)

    W1b_x, W1b_a = W1b[:d], W1b[d:]
    W2b_h, W2b_a = W2b[:d1], W2b[d1:]

    m1 = _tc_table(x_pad, W1a, b1a, n_pad, blk)
    acc1, cnt = _sc_aggregate(n_pad, d1, k_chunks, True)(
        m1, src_r, dst_r, zeros1, zeros_c, ones)
    h1, m2 = _tc_update(x_pad, acc1, cnt, W1b_x, W1b_a, b1b, W2a, b2a,
                        n_pad, blk)
    (acc2,) = _sc_aggregate(n_pad, d2, k_chunks, False)(m2, src_r, dst_r,
                                                        zeros2)
    h2, _ = _tc_update(h1, acc2, cnt, W2b_h, W2b_a, b2b, None, None,
                       n_pad, blk)

    def final_body(h_ref, w_ref, b_ref, o_ref):
        o_ref[...] = jnp.dot(h_ref[...], w_ref[...],
                             preferred_element_type=F32) + b_ref[...]

    y = pl.pallas_call(
        final_body,
        grid=(n_pad // blk,),
        in_specs=[
            pl.BlockSpec((blk, d2), lambda i: (i, 0)),
            pl.BlockSpec((d2, 1), lambda i: (0, 0)),
            pl.BlockSpec((1, 1), lambda i: (0, 0)),
        ],
        out_specs=pl.BlockSpec((blk, 1), lambda i: (i, 0)),
        out_shape=jax.ShapeDtypeStruct((n_pad, 1), F32),
    )(h2, Wl, bl.reshape(1, 1))
    return y[:n]
